# Initial kernel scaffold; baseline (speedup 1.0000x reference)
import jax
import jax.numpy as jnp
from jax.experimental import pallas as pl


def kernel(z_feats, codebook):
    raise NotImplementedError("write your pallas kernel here")


# speedup vs baseline: 1.2553x; 1.2553x over previous
import functools

import jax
import jax.numpy as jnp
from jax import lax
from jax.experimental import pallas as pl
from jax.experimental.pallas import tpu as pltpu
from jax.experimental.pallas import tpu_sc as plsc

_NE = 8192
_D = 256
_NV = 16384

_BZ = 1024
_BC = 2048
_NZB = _NV // _BZ
_NCB = _NE // _BC


def _dist_argmin_body(a2_ref, b2_ref, zb_ref, cbt_ref, idx_ref, loss_ref,
                      rmin_ref, ridx_ref):
    i = pl.program_id(0)
    j = pl.program_id(1)
    ab = lax.dot_general(zb_ref[...], cbt_ref[...], (((1,), (0,)), ((), ())),
                         preferred_element_type=jnp.float32)
    t1 = a2_ref[...][:, None] + b2_ref[...][None, :]
    d2 = t1 - 2.0 * ab
    m = jnp.min(d2, axis=1)
    iota = lax.broadcasted_iota(jnp.int32, (_BZ, _BC), 1)
    local = jnp.min(jnp.where(d2 == m[:, None], iota, _BC), axis=1)
    gidx = local + j * _BC

    @pl.when(j == 0)
    def _():
        rmin_ref[...] = m
        ridx_ref[...] = gidx

    @pl.when(j > 0)
    def _():
        prev = rmin_ref[...]
        better = m < prev
        ridx_ref[...] = jnp.where(better, gidx, ridx_ref[...])
        rmin_ref[...] = jnp.where(better, m, prev)

    @pl.when(j == _NCB - 1)
    def _():
        idx_ref[...] = ridx_ref[...]
        s = jnp.sum(rmin_ref[...])
        prev = jnp.where(i == 0, 0.0, loss_ref[0, 0])
        loss_ref[0, 0] = prev + s


_dist_argmin = pl.pallas_call(
    _dist_argmin_body,
    grid=(_NZB, _NCB),
    in_specs=[
        pl.BlockSpec((_BZ,), lambda i, j: (i,)),
        pl.BlockSpec((_BC,), lambda i, j: (j,)),
        pl.BlockSpec((_BZ, _D), lambda i, j: (i, 0)),
        pl.BlockSpec((_D, _BC), lambda i, j: (0, j)),
    ],
    out_specs=[
        pl.BlockSpec((_BZ,), lambda i, j: (i,)),
        pl.BlockSpec(memory_space=pltpu.SMEM, block_shape=(1, 1),
                     index_map=lambda i, j: (0, 0)),
    ],
    out_shape=[
        jax.ShapeDtypeStruct((_NV,), jnp.int32),
        jax.ShapeDtypeStruct((1, 1), jnp.float32),
    ],
    scratch_shapes=[
        pltpu.VMEM((_BZ,), jnp.float32),
        pltpu.VMEM((_BZ,), jnp.int32),
    ],
)

_NW = 32
_BPW = _NV // _NW
_CH = 128
_NCH = _BPW // _CH

@functools.cache
def _make_gather_rows():
    mesh = plsc.VectorSubcoreMesh(core_axis_name="c", subcore_axis_name="s")

    @functools.partial(
        pl.kernel,
        mesh=mesh,
        out_type=jax.ShapeDtypeStruct((_NV, _D), jnp.float32),
        scratch_types=[
            pltpu.VMEM((_CH,), jnp.int32),
            pltpu.VMEM((_CH, _D), jnp.float32),
            pltpu.SemaphoreType.DMA,
        ],
    )
    def _gather_rows(cb_hbm, idx_hbm, out_hbm, idx_v, rows_v, sem):
        wid = lax.axis_index("s") * 2 + lax.axis_index("c")
        base = wid * _BPW
        for ci in range(_NCH):
            off = base + ci * _CH
            pltpu.sync_copy(idx_hbm.at[pl.ds(off, _CH)], idx_v)
            pltpu.async_copy(cb_hbm.at[idx_v], rows_v, sem).wait()
            pltpu.sync_copy(rows_v, out_hbm.at[pl.ds(off, _CH)])

    return _gather_rows


def kernel(z_feats, codebook):
    a2 = jnp.sum(z_feats * z_feats, axis=1)
    b2 = jnp.sum(codebook * codebook, axis=1)
    idx, loss_sum = _dist_argmin(a2, b2, z_feats, codebook.T)
    quantized = _make_gather_rows()(codebook, idx)
    loss = loss_sum[0, 0] / jnp.float32(_NV * _D)
    enc = idx.astype(jnp.float32)[:, None]
    return quantized, loss, loss, enc


# speedup vs baseline: 1.4050x; 1.1193x over previous
import functools

import jax
import jax.numpy as jnp
from jax import lax
from jax.experimental import pallas as pl
from jax.experimental.pallas import tpu as pltpu
from jax.experimental.pallas import tpu_sc as plsc

_NE = 8192
_D = 256
_NV = 16384

_BZ = 1024
_BC = 2048
_NZB = _NV // _BZ
_NCB = _NE // _BC


_LANES = 128
_KS = _BC // _LANES


def _dist_argmin_body(a2_ref, b2_ref, zb_ref, cbt_ref, idx_ref, loss_ref,
                      pmin_ref, pblk_ref):
    i = pl.program_id(0)
    j = pl.program_id(1)
    ab = lax.dot_general(zb_ref[...], cbt_ref[...], (((1,), (0,)), ((), ())),
                         preferred_element_type=jnp.float32)
    a2c = a2_ref[...][:, None]
    b2 = b2_ref[...]

    def strips(pm, pb):
        for k in range(_KS):
            lo, hi = k * _LANES, (k + 1) * _LANES
            t1 = a2c + b2[lo:hi][None, :]
            d2 = t1 - 2.0 * ab[:, lo:hi]
            blk = jnp.full((_BZ, _LANES), j * _KS + k, jnp.int32)
            better = d2 < pm
            pb = jnp.where(better, blk, pb)
            pm = jnp.minimum(d2, pm)
        return pm, pb

    @pl.when(j == 0)
    def _():
        pm = jnp.full((_BZ, _LANES), jnp.inf, jnp.float32)
        pb = jnp.zeros((_BZ, _LANES), jnp.int32)
        pm, pb = strips(pm, pb)
        pmin_ref[...] = pm
        pblk_ref[...] = pb

    @pl.when(j > 0)
    def _():
        pm, pb = strips(pmin_ref[...], pblk_ref[...])
        pmin_ref[...] = pm
        pblk_ref[...] = pb

    @pl.when(j == _NCB - 1)
    def _():
        pm = pmin_ref[...]
        gidx = pblk_ref[...] * _LANES + lax.broadcasted_iota(
            jnp.int32, (_BZ, _LANES), 1)
        m = jnp.min(pm, axis=1)
        amin = jnp.min(jnp.where(pm == m[:, None], gidx, _NE), axis=1)
        idx_ref[...] = amin
        s = jnp.sum(m)
        prev = jnp.where(i == 0, 0.0, loss_ref[0, 0])
        loss_ref[0, 0] = prev + s


_dist_argmin = pl.pallas_call(
    _dist_argmin_body,
    grid=(_NZB, _NCB),
    in_specs=[
        pl.BlockSpec((_BZ,), lambda i, j: (i,)),
        pl.BlockSpec((_BC,), lambda i, j: (j,)),
        pl.BlockSpec((_BZ, _D), lambda i, j: (i, 0)),
        pl.BlockSpec((_D, _BC), lambda i, j: (0, j)),
    ],
    out_specs=[
        pl.BlockSpec((_BZ,), lambda i, j: (i,)),
        pl.BlockSpec(memory_space=pltpu.SMEM, block_shape=(1, 1),
                     index_map=lambda i, j: (0, 0)),
    ],
    out_shape=[
        jax.ShapeDtypeStruct((_NV,), jnp.int32),
        jax.ShapeDtypeStruct((1, 1), jnp.float32),
    ],
    scratch_shapes=[
        pltpu.VMEM((_BZ, _LANES), jnp.float32),
        pltpu.VMEM((_BZ, _LANES), jnp.int32),
    ],
)

_NW = 32
_BPW = _NV // _NW
_CH = 128
_NCH = _BPW // _CH

@functools.cache
def _make_gather_rows():
    mesh = plsc.VectorSubcoreMesh(core_axis_name="c", subcore_axis_name="s")

    @functools.partial(
        pl.kernel,
        mesh=mesh,
        out_type=jax.ShapeDtypeStruct((_NV, _D), jnp.float32),
        scratch_types=[
            pltpu.VMEM((_CH,), jnp.int32),
            pltpu.VMEM((_CH, _D), jnp.float32),
            pltpu.SemaphoreType.DMA,
        ],
    )
    def _gather_rows(cb_hbm, idx_hbm, out_hbm, idx_v, rows_v, sem):
        wid = lax.axis_index("s") * 2 + lax.axis_index("c")
        base = wid * _BPW
        for ci in range(_NCH):
            off = base + ci * _CH
            pltpu.sync_copy(idx_hbm.at[pl.ds(off, _CH)], idx_v)
            pltpu.async_copy(cb_hbm.at[idx_v], rows_v, sem).wait()
            pltpu.sync_copy(rows_v, out_hbm.at[pl.ds(off, _CH)])

    return _gather_rows


def kernel(z_feats, codebook):
    a2 = jnp.sum(z_feats * z_feats, axis=1)
    b2 = jnp.sum(codebook * codebook, axis=1)
    idx, loss_sum = _dist_argmin(a2, b2, z_feats, codebook.T)
    quantized = _make_gather_rows()(codebook, idx)
    loss = loss_sum[0, 0] / jnp.float32(_NV * _D)
    enc = idx.astype(jnp.float32)[:, None]
    return quantized, loss, loss, enc
